# baseline (speedup 1.0000x reference)
import functools

import jax
import jax.numpy as jnp
from jax import lax
from jax.experimental import pallas as pl
from jax.experimental.pallas import tpu as pltpu
from jax.experimental.pallas import tpu_sc as plsc

D = 64
L = 200
B = 4096
NC, NS = 2, 16
NW = NC * NS
R = B // NW
LP = 208
NV = LP // 16


def _sc_body(x_hbm, w_hbm, b_hbm, table_hbm, out_hbm,
             idx_v, w_v, b_v, shr, rows0, rows1, out_v, sem0, sem1, semb):
    cid = lax.axis_index("c")
    sid = lax.axis_index("s")
    wid = sid * NC + cid
    base = wid * R

    pltpu.sync_copy(w_hbm, w_v)
    pltpu.sync_copy(b_hbm, b_v)
    pltpu.sync_copy(x_hbm.at[pl.ds(base, R)], idx_v)

    b_vec = b_v[pl.ds(0, 16)]
    lanes = lax.broadcasted_iota(jnp.int32, (16,), 0)

    def allreduce16(v):
        dnums = lax.GatherDimensionNumbers(
            offset_dims=(), collapsed_slice_dims=(0,), start_index_map=(0,))
        for k in (1, 2, 4, 8):
            perm = jnp.bitwise_xor(lanes, k)
            v = v + lax.gather(v, perm[:, None], dnums, slice_sizes=(1,),
                               mode=lax.GatherScatterMode.PROMISE_IN_BOUNDS)
        return v

    def fire(r, sem):
        for c in range(NV):
            v = idx_v[r, pl.ds(c * 16, 16)]
            for j in range(16):
                rr = v[j]
                pltpu.make_async_copy(
                    table_hbm.at[rr], shr.at[sid, c * 16 + j], sem).start()

    def drain(sem):
        pltpu.make_async_copy(
            table_hbm.at[pl.ds(0, LP)], shr.at[sid], sem).wait()

    def compute(rows_buf):
        def body_i(i, accs):
            a = list(accs)
            for k in range(4):
                rv = rows_buf[i, pl.ds(k * 16, 16)]
                wv = w_v[i, pl.ds(k * 16, 16)]
                a[k] = a[k] + rv * wv
            return tuple(a)

        init = tuple(jnp.zeros((16,), jnp.float32) for _ in range(4))
        a0, a1, a2, a3 = lax.fori_loop(0, LP, body_i, init)
        tot = (a0 + a1) + (a2 + a3)
        return allreduce16(tot) + b_vec

    bufs = (rows0, rows1)
    NB = 2
    fire(0, sem0)

    def loop_body(t, vec):
        g = t * NB
        for p in range(NB):
            r = g + p
            drain(sem0)
            pltpu.sync_copy(shr.at[sid], bufs[p])
            fire(lax.min(r + 1, R - 1), sem0)
            s = compute(bufs[p])
            vec = jnp.where(lanes == (r % 16), s, vec)

        @pl.when(t % (16 // NB) == (16 // NB) - 1)
        def _():
            out_v[pl.ds(g + NB - 16, 16)] = vec

        return vec

    lax.fori_loop(0, R // NB, loop_body, jnp.zeros((16,), jnp.float32))

    drain(sem0)

    pltpu.sync_copy(out_v, out_hbm.at[pl.ds(base, R)])


@functools.partial(jax.jit, static_argnames=())
def kernel(x, table, W, b):
    x2 = jnp.pad(x.astype(jnp.int32), ((0, 0), (0, LP - L)))
    w2 = jnp.pad(W.reshape(L, D).astype(jnp.float32), ((0, LP - L), (0, 0)))
    b16 = jnp.broadcast_to(b.astype(jnp.float32), (16,))

    mesh = plsc.VectorSubcoreMesh(core_axis_name="c", subcore_axis_name="s")
    call = functools.partial(
        pl.kernel,
        mesh=mesh,
        out_type=jax.ShapeDtypeStruct((B,), jnp.float32),
        scratch_types=[
            pltpu.VMEM((R, LP), jnp.int32),
            pltpu.VMEM((LP, D), jnp.float32),
            pltpu.VMEM((16,), jnp.float32),
            pltpu.VMEM_SHARED((NS, LP, D), jnp.float32),
            pltpu.VMEM((LP, D), jnp.float32),
            pltpu.VMEM((LP, D), jnp.float32),
            pltpu.VMEM((R,), jnp.float32),
            pltpu.SemaphoreType.DMA,
            pltpu.SemaphoreType.DMA,
            pltpu.SemaphoreType.DMA,
        ],
    )(_sc_body)
    return call(x2, w2, b16, table)


# speedup vs baseline: 1.5177x; 1.5177x over previous
import functools

import jax
import jax.numpy as jnp
from jax import lax
from jax.experimental import pallas as pl
from jax.experimental.pallas import tpu as pltpu
from jax.experimental.pallas import tpu_sc as plsc

D = 64
L = 200
B = 4096
NC, NS = 2, 16
NW = NC * NS
R = B // NW
LP = 208
NV = LP // 16


def _sc_body(x_hbm, w_hbm, b_hbm, table_hbm, out_hbm,
             idx_v, w_v, b_v, rows0, rows1, rows2, rows3, out_v,
             sem0, sem1, sem2, sem3):
    cid = lax.axis_index("c")
    sid = lax.axis_index("s")
    wid = sid * NC + cid
    base = wid * R

    pltpu.sync_copy(w_hbm, w_v)
    pltpu.sync_copy(b_hbm, b_v)
    pltpu.sync_copy(x_hbm.at[pl.ds(base, R)], idx_v)

    b_vec = b_v[pl.ds(0, 16)]
    lanes = lax.broadcasted_iota(jnp.int32, (16,), 0)

    def allreduce16(v):
        dnums = lax.GatherDimensionNumbers(
            offset_dims=(), collapsed_slice_dims=(0,), start_index_map=(0,))
        for k in (1, 2, 4, 8):
            perm = jnp.bitwise_xor(lanes, k)
            v = v + lax.gather(v, perm[:, None], dnums, slice_sizes=(1,),
                               mode=lax.GatherScatterMode.PROMISE_IN_BOUNDS)
        return v

    def fire(r, rows_buf, sem):
        for c in range(NV):
            vec = idx_v[r, pl.ds(c * 16, 16)]
            pltpu.make_async_copy(
                table_hbm.at[vec], rows_buf.at[pl.ds(c * 16, 16)], sem).start()

    def drain(r, rows_buf, sem):
        for c in range(NV):
            vec = idx_v[r, pl.ds(c * 16, 16)]
            pltpu.make_async_copy(
                table_hbm.at[vec], rows_buf.at[pl.ds(c * 16, 16)], sem).wait()

    def compute(rows_buf):
        def body_i(i, accs):
            a = list(accs)
            for k in range(4):
                rv = rows_buf[i, pl.ds(k * 16, 16)]
                wv = w_v[i, pl.ds(k * 16, 16)]
                a[k] = a[k] + rv * wv
            return tuple(a)

        init = tuple(jnp.zeros((16,), jnp.float32) for _ in range(4))
        a0, a1, a2, a3 = lax.fori_loop(0, LP, body_i, init)
        tot = (a0 + a1) + (a2 + a3)
        return allreduce16(tot) + b_vec

    bufs = (rows0, rows1, rows2, rows3)
    sems = (sem0, sem1, sem2, sem3)
    NB = 4
    for p in range(NB):
        fire(p, bufs[p], sems[p])

    def loop_body(t, vec):
        g = t * NB
        for p in range(NB):
            r = g + p
            drain(r, bufs[p], sems[p])
            s = compute(bufs[p])
            fire(lax.min(r + NB, R - 1), bufs[p], sems[p])
            vec = jnp.where(lanes == (r % 16), s, vec)

        @pl.when(t % (16 // NB) == (16 // NB) - 1)
        def _():
            out_v[pl.ds(g + NB - 16, 16)] = vec

        return vec

    lax.fori_loop(0, R // NB, loop_body, jnp.zeros((16,), jnp.float32))

    for p in range(NB):
        drain(R - 1, bufs[p], sems[p])

    pltpu.sync_copy(out_v, out_hbm.at[pl.ds(base, R)])


@functools.partial(jax.jit, static_argnames=())
def kernel(x, table, W, b):
    x2 = jnp.pad(x.astype(jnp.int32), ((0, 0), (0, LP - L)))
    w2 = jnp.pad(W.reshape(L, D).astype(jnp.float32), ((0, LP - L), (0, 0)))
    b16 = jnp.broadcast_to(b.astype(jnp.float32), (16,))

    mesh = plsc.VectorSubcoreMesh(core_axis_name="c", subcore_axis_name="s")
    call = functools.partial(
        pl.kernel,
        mesh=mesh,
        out_type=jax.ShapeDtypeStruct((B,), jnp.float32),
        compiler_params=pltpu.CompilerParams(use_tc_tiling_on_sc=False),
        scratch_types=[
            pltpu.VMEM((R, LP), jnp.int32),
            pltpu.VMEM((LP, D), jnp.float32),
            pltpu.VMEM((16,), jnp.float32),
            pltpu.VMEM((LP, D), jnp.float32),
            pltpu.VMEM((LP, D), jnp.float32),
            pltpu.VMEM((LP, D), jnp.float32),
            pltpu.VMEM((LP, D), jnp.float32),
            pltpu.VMEM((R,), jnp.float32),
            pltpu.SemaphoreType.DMA,
            pltpu.SemaphoreType.DMA,
            pltpu.SemaphoreType.DMA,
            pltpu.SemaphoreType.DMA,
        ],
    )(_sc_body)
    return call(x2, w2, b16, table)
